# baseline (speedup 1.0000x reference)
import jax, jax.numpy as jnp

def kernel(features, edge_index, initial_features, W_a):
    src = edge_index[0]; dst = edge_index[1]
    E = src.shape[0]; N = features.shape[0]
    degs = jax.ops.segment_sum(jnp.ones((E,), jnp.float32), dst, num_segments=N)
    degs = jnp.clip(degs, 1.0, None)
    norm = jnp.power(degs, -0.5)[:, None]
    h = features * norm
    m = jnp.take(h, src, axis=0)
    h_agg = jax.ops.segment_sum(m, dst, num_segments=N)
    h = h_agg * norm
    hs = jnp.concatenate([features, h], axis=1)
    alpha = jax.nn.sigmoid(hs @ W_a.T)
    return alpha * h + initial_features


# speedup vs baseline: 2.0924x; 2.0924x over previous
import functools

import jax
import jax.numpy as jnp
from jax import lax
from jax.experimental import pallas as pl
from jax.experimental.pallas import tpu as pltpu
from jax.experimental.pallas import tpu_sc as plsc

N = 10000
E = 160000
D = 256
NC, NS, L = 2, 16, 16
NW = NC * NS

NPAD = 10240
EP = 163840
EPT1 = EP // NW
OWN = 313
ACCR = OWN + 1
SEC = 2048
NSEC = EP // SEC
K = 64
PCAP = SEC + 2 * K

_mesh = plsc.VectorSubcoreMesh(
    core_axis_name="c", subcore_axis_name="s", num_cores=NC, num_subcores=NS
)
_sc_params = pltpu.CompilerParams(needs_layout_passes=False)


def _hist_body(dst32, hists, dsth, hist):
    c = lax.axis_index("c")
    s = lax.axis_index("s")
    w = c * NS + s

    def _zero(i, _):
        hist[pl.ds(i * L, L)] = jnp.zeros((L,), jnp.float32)
        return 0
    lax.fori_loop(0, NPAD // L, _zero, 0)

    pltpu.sync_copy(dst32.at[w], dsth)

    def _count(i, _):
        d = dsth[pl.ds(i * L, L)]
        cnt, last = plsc.scan_count(d)
        val = cnt.astype(jnp.float32)
        plsc.addupdate_scatter(hist, [d], val, mask=last)
        return 0
    lax.fori_loop(0, EPT1 // L, _count, 0)

    pltpu.sync_copy(hist, hists.at[w])


_hist_kernel = functools.partial(
    pl.kernel,
    out_type=jax.ShapeDtypeStruct((NW, NPAD), jnp.float32),
    mesh=_mesh,
    scratch_types=[
        pltpu.VMEM((EPT1,), jnp.int32),
        pltpu.VMEM((NPAD,), jnp.float32),
    ],
    compiler_params=_sc_params,
)(_hist_body)


def _mp_body(g_hbm, srcF, dstF, zflat, aggp, rawS, rawD, Ps, Pd, sidx, rows,
             acc, sem):
    c = lax.axis_index("c")
    s = lax.axis_index("s")
    w = c * NS + s
    lo = w * OWN

    zn = zflat.shape[0]
    nfull = (ACCR * D) // zn
    for r in range(nfull):
        pltpu.sync_copy(zflat, acc.at[pl.ds(r * zn, zn)])
    rem = ACCR * D - nfull * zn
    if rem:
        pltpu.sync_copy(zflat.at[pl.ds(0, rem)], acc.at[pl.ds(nfull * zn, rem)])

    def _do_batch(h):
        hh = pl.multiple_of(h, 8)

        def _stage(j, _):
            sidx[pl.ds(j * L, L)] = Ps[pl.ds(hh + j * L, L)]
            return 0
        lax.fori_loop(0, K // L, _stage, 0)
        pltpu.async_copy(g_hbm.at[sidx], rows, sem).wait()

        def _grp(g, _):
            dstv = Pd[pl.ds(hh + g * L, L)] * D
            for l in range(L):
                base = dstv[l]
                for kk in range(D // L):
                    x = rows[g * L + l, pl.ds(kk * L, L)]
                    plsc.addupdate(acc.at[pl.ds(base + kk * L, L)], x)
            return 0
        lax.fori_loop(0, K // L, _grp, 0)

    def _section(sec, carry):
        h0, t0 = carry
        pltpu.sync_copy(srcF.at[pl.ds(sec * SEC, SEC)], rawS)
        pltpu.sync_copy(dstF.at[pl.ds(sec * SEC, SEC)], rawD)

        def _scan(i, t):
            d = rawD[pl.ds(i * L, L)]
            sv = rawS[pl.ds(i * L, L)]
            m = (d >= lo) & (d < lo + OWN)
            plsc.store_compressed(Pd.at[pl.ds(t, L)], d - lo, mask=m)
            plsc.store_compressed(Ps.at[pl.ds(t, L)], sv, mask=m)
            return t + jnp.sum(m.astype(jnp.int32))
        t = lax.fori_loop(0, SEC // L, _scan, t0)

        def _cond(ht):
            h, t = ht
            return t - h >= K

        def _w(ht):
            h, t = ht
            _do_batch(h)
            return h + K, t
        h, t = lax.while_loop(_cond, _w, (h0, t))

        hh = pl.multiple_of(h, 8)

        def _shift(j, _):
            Ps[pl.ds(j * L, L)] = Ps[pl.ds(hh + j * L, L)]
            Pd[pl.ds(j * L, L)] = Pd[pl.ds(hh + j * L, L)]
            return 0
        lax.fori_loop(0, K // L, _shift, 0)
        return 0, t - h

    _, t = lax.fori_loop(0, NSEC, _section, (0, 0))

    @pl.when(t > 0)
    def _tail():
        full = lax.full((L,), True)

        def _pad(j, _):
            plsc.store_compressed(Ps.at[pl.ds(t + j * L, L)],
                                  jnp.zeros((L,), jnp.int32), mask=full)
            plsc.store_compressed(Pd.at[pl.ds(t + j * L, L)],
                                  jnp.full((L,), OWN, jnp.int32), mask=full)
            return 0
        lax.fori_loop(0, K // L, _pad, 0)
        _do_batch(0)

    pltpu.sync_copy(acc.at[pl.ds(0, OWN * D)],
                    aggp.at[pl.ds(w * OWN * D, OWN * D)])


_mp_kernel = functools.partial(
    pl.kernel,
    out_type=jax.ShapeDtypeStruct((NW * OWN * D,), jnp.float32),
    mesh=_mesh,
    scratch_types=[
        pltpu.VMEM((SEC,), jnp.int32),
        pltpu.VMEM((SEC,), jnp.int32),
        pltpu.VMEM((PCAP,), jnp.int32),
        pltpu.VMEM((PCAP,), jnp.int32),
        pltpu.VMEM((K,), jnp.int32),
        pltpu.VMEM((K, D), jnp.float32),
        pltpu.VMEM((ACCR * D,), jnp.float32),
        pltpu.SemaphoreType.DMA,
    ],
    compiler_params=_sc_params,
)(_mp_body)


def _norm_body(h_ref, n_ref):
    d = jnp.sum(h_ref[...], axis=0)
    n_ref[...] = lax.rsqrt(jnp.maximum(d, 1.0))


def _norm(hists):
    blk = 2048
    return pl.pallas_call(
        _norm_body,
        grid=(NPAD // blk,),
        in_specs=[pl.BlockSpec((NW, blk), lambda i: (0, i))],
        out_specs=pl.BlockSpec((blk,), lambda i: (i,)),
        out_shape=jax.ShapeDtypeStruct((NPAD,), jnp.float32),
    )(hists)


def _scale_body(f_ref, n_ref, g_ref):
    g_ref[...] = f_ref[...] * n_ref[...]


def _scale(features, norm2):
    blk = 2000
    return pl.pallas_call(
        _scale_body,
        grid=(N // blk,),
        in_specs=[
            pl.BlockSpec((blk, D), lambda i: (i, 0)),
            pl.BlockSpec((blk, 1), lambda i: (i, 0)),
        ],
        out_specs=pl.BlockSpec((blk, D), lambda i: (i, 0)),
        out_shape=jax.ShapeDtypeStruct((N, D), jnp.float32),
    )(features, norm2)


def _gate_body(a_ref, n_ref, f_ref, i_ref, w1_ref, w2_ref, o_ref):
    h = a_ref[...] * n_ref[...]
    s1 = jnp.sum(f_ref[...] * w1_ref[...], axis=1, keepdims=True)
    s2 = jnp.sum(h * w2_ref[...], axis=1, keepdims=True)
    alpha = jax.nn.sigmoid(s1 + s2)
    o_ref[...] = alpha * h + i_ref[...]


def _gate(agg2, norm2, features, initial, w1, w2):
    blk = 2000
    return pl.pallas_call(
        _gate_body,
        grid=(N // blk,),
        in_specs=[
            pl.BlockSpec((blk, D), lambda i: (i, 0)),
            pl.BlockSpec((blk, 1), lambda i: (i, 0)),
            pl.BlockSpec((blk, D), lambda i: (i, 0)),
            pl.BlockSpec((blk, D), lambda i: (i, 0)),
            pl.BlockSpec((1, D), lambda i: (0, 0)),
            pl.BlockSpec((1, D), lambda i: (0, 0)),
        ],
        out_specs=pl.BlockSpec((blk, D), lambda i: (i, 0)),
        out_shape=jax.ShapeDtypeStruct((N, D), jnp.float32),
    )(agg2, norm2, features, initial, w1, w2)


def kernel(features, edge_index, initial_features, W_a):
    assert features.shape == (N, D) and edge_index.shape == (2, E)
    src = edge_index[0]
    dst = edge_index[1]

    pad_n = EP - E
    fill = jnp.arange(pad_n, dtype=jnp.int32)
    src_p = jnp.concatenate([src, fill % N])
    dst_p = jnp.concatenate([dst, N + fill % (NPAD - N)])
    dst32 = dst_p.reshape(NW, EPT1)

    hists = _hist_kernel(dst32)
    norm2 = _norm(hists).reshape(NPAD, 1)
    g = _scale(features, norm2)

    zflat = jnp.zeros((16384,), jnp.float32)
    aggf = _mp_kernel(g, src_p, dst_p, zflat)
    agg2 = aggf.reshape(NW * OWN, D)

    w1 = W_a[:, :D]
    w2 = W_a[:, D:]
    return _gate(agg2, norm2, features, initial_features, w1, w2)


# speedup vs baseline: 2.7728x; 1.3252x over previous
import functools

import jax
import jax.numpy as jnp
from jax import lax
from jax.experimental import pallas as pl
from jax.experimental.pallas import tpu as pltpu
from jax.experimental.pallas import tpu_sc as plsc

N = 10000
E = 160000
D = 256
NC, NS, L = 2, 16, 16
NW = NC * NS

NPAD = 10240
EP = 163840
EPT1 = EP // NW
OWN = 313
ACCR = OWN + 1
SEC = 2048
NSEC = EP // SEC
K = 64
PCAP = SEC + 2 * K

_mesh = plsc.VectorSubcoreMesh(
    core_axis_name="c", subcore_axis_name="s", num_cores=NC, num_subcores=NS
)
_sc_params = pltpu.CompilerParams(needs_layout_passes=False)


def _hist_body(dst32, hists, dsth, hist):
    c = lax.axis_index("c")
    s = lax.axis_index("s")
    w = c * NS + s

    def _zero(i, _):
        hist[pl.ds(i * L, L)] = jnp.zeros((L,), jnp.float32)
        return 0
    lax.fori_loop(0, NPAD // L, _zero, 0)

    pltpu.sync_copy(dst32.at[w], dsth)

    def _count(i, _):
        d = dsth[pl.ds(i * L, L)]
        cnt, last = plsc.scan_count(d)
        val = cnt.astype(jnp.float32)
        plsc.addupdate_scatter(hist, [d], val, mask=last)
        return 0
    lax.fori_loop(0, EPT1 // L, _count, 0)

    pltpu.sync_copy(hist, hists.at[w])


_hist_kernel = functools.partial(
    pl.kernel,
    out_type=jax.ShapeDtypeStruct((NW, NPAD), jnp.float32),
    mesh=_mesh,
    scratch_types=[
        pltpu.VMEM((EPT1,), jnp.int32),
        pltpu.VMEM((NPAD,), jnp.float32),
    ],
    compiler_params=_sc_params,
)(_hist_body)


def _mp_body(g_hbm, srcF, dstF, zflat, aggp, rawS, rawD, Ps, Pd, sidx, didx,
             rows, acc, sem, sem2):
    c = lax.axis_index("c")
    s = lax.axis_index("s")
    w = c * NS + s
    lo = w * OWN

    zn = zflat.shape[0]
    nfull = (ACCR * D) // zn
    for r in range(nfull):
        pltpu.sync_copy(zflat, acc.at[pl.ds(r * zn, zn)])
    rem = ACCR * D - nfull * zn
    if rem:
        pltpu.sync_copy(zflat.at[pl.ds(0, rem)], acc.at[pl.ds(nfull * zn, rem)])

    def _start_batch(h, b):
        hh = pl.multiple_of(h, 8)
        p = (b % 2) * K

        def _stage(j, _):
            sidx[pl.ds(p + j * L, L)] = Ps[pl.ds(hh + j * L, L)]
            didx[pl.ds(p + j * L, L)] = Pd[pl.ds(hh + j * L, L)] * D
            return 0
        lax.fori_loop(0, K // L, _stage, 0)
        pltpu.async_copy(g_hbm.at[sidx.at[pl.ds(p, K)]],
                         rows.at[pl.ds(p, K)], sem)

    def _finish_batch(b):
        p = (b % 2) * K
        pltpu.make_async_copy(g_hbm.at[sidx.at[pl.ds(p, K)]],
                              rows.at[pl.ds(p, K)], sem).wait()

        def _grp(g, _):
            dstv = didx[pl.ds(p + g * L, L)]
            for l in range(L):
                base = dstv[l]
                for kk in range(D // L):
                    x = rows[p + g * L + l, pl.ds(kk * L, L)]
                    plsc.addupdate(acc.at[pl.ds(base + kk * L, L)], x)
            return 0
        lax.fori_loop(0, K // L, _grp, 0)

    def _sec_load(sec, q):
        pltpu.async_copy(srcF.at[pl.ds(sec * SEC, SEC)],
                         rawS.at[pl.ds(q * SEC, SEC)], sem2)
        pltpu.async_copy(dstF.at[pl.ds(sec * SEC, SEC)],
                         rawD.at[pl.ds(q * SEC, SEC)], sem2)

    def _sec_wait(q):
        pltpu.make_async_copy(srcF.at[pl.ds(0, SEC)],
                              rawS.at[pl.ds(q * SEC, SEC)], sem2).wait()
        pltpu.make_async_copy(dstF.at[pl.ds(0, SEC)],
                              rawD.at[pl.ds(q * SEC, SEC)], sem2).wait()

    _sec_load(0, 0)

    def _section(sec, carry):
        h0, t0, b0 = carry
        q = sec % 2
        _sec_wait(q)

        @pl.when(sec + 1 < NSEC)
        def _():
            _sec_load(sec + 1, 1 - q)

        qb = q * SEC

        def _scan(i, t):
            d = rawD[pl.ds(qb + i * L, L)]
            sv = rawS[pl.ds(qb + i * L, L)]
            m = (d >= lo) & (d < lo + OWN)
            plsc.store_compressed(Pd.at[pl.ds(t, L)], d - lo, mask=m)
            plsc.store_compressed(Ps.at[pl.ds(t, L)], sv, mask=m)
            return t + plsc.all_reduce_population_count(m)[0]
        t = lax.fori_loop(0, SEC // L, _scan, t0)

        def _cond(htb):
            h, t, b = htb
            return t - h >= K

        def _w(htb):
            h, t, b = htb
            _start_batch(h, b)

            @pl.when(b > 0)
            def _():
                _finish_batch(b - 1)
            return h + K, t, b + 1
        h, t, b = lax.while_loop(_cond, _w, (h0, t, b0))

        hh = pl.multiple_of(h, 8)

        def _shift(j, _):
            Ps[pl.ds(j * L, L)] = Ps[pl.ds(hh + j * L, L)]
            Pd[pl.ds(j * L, L)] = Pd[pl.ds(hh + j * L, L)]
            return 0
        lax.fori_loop(0, K // L, _shift, 0)
        return 0, t - h, b

    _, t, b = lax.fori_loop(0, NSEC, _section, (0, 0, 0))

    @pl.when(t > 0)
    def _tail():
        full = lax.full((L,), True)

        def _pad(j, _):
            plsc.store_compressed(Ps.at[pl.ds(t + j * L, L)],
                                  jnp.zeros((L,), jnp.int32), mask=full)
            plsc.store_compressed(Pd.at[pl.ds(t + j * L, L)],
                                  jnp.full((L,), OWN, jnp.int32), mask=full)
            return 0
        lax.fori_loop(0, K // L, _pad, 0)
        _start_batch(0, b)

        @pl.when(b > 0)
        def _():
            _finish_batch(b - 1)
        _finish_batch(b)

    @pl.when(t == 0)
    def _drain():
        @pl.when(b > 0)
        def _():
            _finish_batch(b - 1)

    pltpu.sync_copy(acc.at[pl.ds(0, OWN * D)],
                    aggp.at[pl.ds(w * OWN * D, OWN * D)])


_mp_kernel = functools.partial(
    pl.kernel,
    out_type=jax.ShapeDtypeStruct((NW * OWN * D,), jnp.float32),
    mesh=_mesh,
    scratch_types=[
        pltpu.VMEM((2 * SEC,), jnp.int32),
        pltpu.VMEM((2 * SEC,), jnp.int32),
        pltpu.VMEM((PCAP,), jnp.int32),
        pltpu.VMEM((PCAP,), jnp.int32),
        pltpu.VMEM((2 * K,), jnp.int32),
        pltpu.VMEM((2 * K,), jnp.int32),
        pltpu.VMEM((2 * K, D), jnp.float32),
        pltpu.VMEM((ACCR * D,), jnp.float32),
        pltpu.SemaphoreType.DMA,
        pltpu.SemaphoreType.DMA,
    ],
    compiler_params=_sc_params,
)(_mp_body)


def _norm_body(h_ref, n_ref):
    d = jnp.sum(h_ref[...], axis=0)
    n_ref[...] = lax.rsqrt(jnp.maximum(d, 1.0))


def _norm(hists):
    blk = 2048
    return pl.pallas_call(
        _norm_body,
        grid=(NPAD // blk,),
        in_specs=[pl.BlockSpec((NW, blk), lambda i: (0, i))],
        out_specs=pl.BlockSpec((blk,), lambda i: (i,)),
        out_shape=jax.ShapeDtypeStruct((NPAD,), jnp.float32),
    )(hists)


def _scale_body(f_ref, n_ref, g_ref):
    g_ref[...] = f_ref[...] * n_ref[...]


def _scale(features, norm2):
    blk = 2000
    return pl.pallas_call(
        _scale_body,
        grid=(N // blk,),
        in_specs=[
            pl.BlockSpec((blk, D), lambda i: (i, 0)),
            pl.BlockSpec((blk, 1), lambda i: (i, 0)),
        ],
        out_specs=pl.BlockSpec((blk, D), lambda i: (i, 0)),
        out_shape=jax.ShapeDtypeStruct((N, D), jnp.float32),
    )(features, norm2)


def _gate_body(a_ref, n_ref, f_ref, i_ref, w1_ref, w2_ref, o_ref):
    h = a_ref[...] * n_ref[...]
    s1 = jnp.sum(f_ref[...] * w1_ref[...], axis=1, keepdims=True)
    s2 = jnp.sum(h * w2_ref[...], axis=1, keepdims=True)
    alpha = jax.nn.sigmoid(s1 + s2)
    o_ref[...] = alpha * h + i_ref[...]


def _gate(agg2, norm2, features, initial, w1, w2):
    blk = 2000
    return pl.pallas_call(
        _gate_body,
        grid=(N // blk,),
        in_specs=[
            pl.BlockSpec((blk, D), lambda i: (i, 0)),
            pl.BlockSpec((blk, 1), lambda i: (i, 0)),
            pl.BlockSpec((blk, D), lambda i: (i, 0)),
            pl.BlockSpec((blk, D), lambda i: (i, 0)),
            pl.BlockSpec((1, D), lambda i: (0, 0)),
            pl.BlockSpec((1, D), lambda i: (0, 0)),
        ],
        out_specs=pl.BlockSpec((blk, D), lambda i: (i, 0)),
        out_shape=jax.ShapeDtypeStruct((N, D), jnp.float32),
    )(agg2, norm2, features, initial, w1, w2)


def kernel(features, edge_index, initial_features, W_a):
    assert features.shape == (N, D) and edge_index.shape == (2, E)
    src = edge_index[0]
    dst = edge_index[1]

    pad_n = EP - E
    fill = jnp.arange(pad_n, dtype=jnp.int32)
    src_p = jnp.concatenate([src, fill % N])
    dst_p = jnp.concatenate([dst, N + fill % (NPAD - N)])
    dst32 = dst_p.reshape(NW, EPT1)

    hists = _hist_kernel(dst32)
    norm2 = _norm(hists).reshape(NPAD, 1)
    g = _scale(features, norm2)

    zflat = jnp.zeros((16384,), jnp.float32)
    aggf = _mp_kernel(g, src_p, dst_p, zflat)
    agg2 = aggf.reshape(NW * OWN, D)

    w1 = W_a[:, :D]
    w2 = W_a[:, D:]
    return _gate(agg2, norm2, features, initial_features, w1, w2)
